# baseline (speedup 1.0000x reference)
import functools

import numpy as np

import jax
import jax.numpy as jnp
from jax import lax
from jax.experimental import pallas as pl
from jax.experimental.pallas import tpu as pltpu
from jax.experimental.pallas import tpu_sc as plsc

D = 128
LN = 16
NC = 2
NS = 16
NW = NC * NS
CH = 128
NB = 2
RB = 1024


def _sc_degree(dst3, n_pad):
    cpt = dst3.shape[1]
    rpt = n_pad // NS
    mesh = plsc.VectorSubcoreMesh(core_axis_name="c", subcore_axis_name="s")

    @functools.partial(
        pl.kernel,
        out_type=jax.ShapeDtypeStruct((NC, n_pad), jnp.float32),
        mesh=mesh,
        scratch_types=[
            pltpu.VMEM((cpt, CH), jnp.int32),
            pltpu.VMEM((CH,), jnp.float32),
            pltpu.VMEM((rpt,), jnp.float32),
            pltpu.VMEM_SHARED((n_pad,), jnp.float32),
        ],
        compiler_params=pltpu.CompilerParams(use_tc_tiling_on_sc=False),
    )
    def deg_kernel(dst_hbm, out_hbm, dst_v, ones_v, zbuf, acc):
        cid = lax.axis_index("c")
        sid = lax.axis_index("s")
        wid = sid * NC + cid
        pltpu.sync_copy(dst_hbm.at[wid], dst_v)

        def zb(i, c):
            zbuf[pl.ds(i * LN, LN)] = jnp.zeros((LN,), jnp.float32)
            return c

        lax.fori_loop(0, rpt // LN, zb, 0)

        def ob(i, c):
            ones_v[pl.ds(i * LN, LN)] = jnp.ones((LN,), jnp.float32)
            return c

        lax.fori_loop(0, CH // LN, ob, 0)
        pltpu.sync_copy(zbuf, acc.at[pl.ds(sid * rpt, rpt)])
        plsc.subcore_barrier()

        def body(c, k):
            pltpu.sync_copy(ones_v, acc.at[dst_v.at[c]], add=True)
            return k

        lax.fori_loop(0, cpt, body, 0)
        plsc.subcore_barrier()
        pltpu.sync_copy(acc.at[pl.ds(sid * rpt, rpt)],
                        out_hbm.at[cid, pl.ds(sid * rpt, rpt)])

    return deg_kernel(dst3)


def _sc_scatter(xs, packed2, zeros, n_pad):
    ept = packed2.shape[1]
    cpt = ept // CH
    rpt = n_pad // NS
    mesh = plsc.VectorSubcoreMesh(core_axis_name="c", subcore_axis_name="s")

    @functools.partial(
        pl.kernel,
        out_type=jax.ShapeDtypeStruct((NC, n_pad, D), jnp.float32),
        mesh=mesh,
        scratch_types=[
            pltpu.VMEM((ept,), jnp.int32),
            pltpu.VMEM((NB * CH,), jnp.int32),
            pltpu.VMEM((NB, CH), jnp.int32),
            [pltpu.VMEM((CH, D), jnp.float32)] * NB,
            pltpu.VMEM_SHARED((n_pad, D), jnp.float32),
            [pltpu.SemaphoreType.DMA] * NB,
            [pltpu.SemaphoreType.DMA] * NB,
        ],
        compiler_params=pltpu.CompilerParams(use_tc_tiling_on_sc=False),
    )
    def scat_kernel(xs_hbm, pk_hbm, zero_hbm, out_hbm,
                    pk_v, src_i, dst_i, bufs, acc, gsem, ssem):
        cid = lax.axis_index("c")
        sid = lax.axis_index("s")
        wid = sid * NC + cid
        r0 = sid * rpt
        pltpu.sync_copy(pk_hbm.at[wid], pk_v)
        pltpu.sync_copy(zero_hbm.at[pl.ds(r0, rpt)], acc.at[pl.ds(r0, rpt)])
        plsc.subcore_barrier()

        def unpack(c, b):
            def ub(u, k):
                p = pk_v[pl.ds(c * CH + u * LN, LN)]
                src_i[pl.ds(b * CH + u * LN, LN)] = p & 0xFFFF
                dst_i[b, pl.ds(u * LN, LN)] = p >> 16
                return k

            lax.fori_loop(0, CH // LN, ub, 0)

        def gather(b):
            return pltpu.make_async_copy(
                xs_hbm.at[src_i.at[pl.ds(b * CH, CH)]], bufs[b], gsem[b])

        def scat_start(b):
            pltpu.async_copy(bufs[b], acc.at[dst_i.at[b]], ssem[b], add=True)

        def scat_wait(b):
            pltpu.make_async_copy(bufs[b], acc.at[dst_i.at[b]], ssem[b]).wait()

        for b in range(NB):
            unpack(b, b)
            gather(b).start()

        def body(j, k):
            c = NB * j
            for b in range(NB):
                gather(b).wait()
                scat_start(b)
            for b in range(NB):
                scat_wait(b)
                unpack(jnp.minimum(c + NB + b, cpt - 1), b)
                gather(b).start()
            return k

        lax.fori_loop(0, cpt // NB, body, 0)
        for b in range(NB):
            gather(b).wait()
        plsc.subcore_barrier()
        pltpu.sync_copy(acc.at[pl.ds(r0, rpt)],
                        out_hbm.at[cid, pl.ds(r0, rpt)])

    return scat_kernel(xs, packed2, zeros)


def _dinv_of(deg_ref):
    deg = deg_ref[0, :] + deg_ref[1, :] + 1.0
    return lax.rsqrt(deg)[:, None]


def _tc_embed1(x_pad, W1, deg_part, n_pad):
    def body(x_ref, w_ref, deg_ref, o_ref):
        xw = jnp.dot(x_ref[...], w_ref[...], preferred_element_type=jnp.float32)
        o_ref[...] = xw * _dinv_of(deg_ref)

    return pl.pallas_call(
        body,
        grid=(n_pad // RB,),
        in_specs=[
            pl.BlockSpec((RB, D), lambda i: (i, 0)),
            pl.BlockSpec((D, D), lambda i: (0, 0)),
            pl.BlockSpec((NC, RB), lambda i: (0, i)),
        ],
        out_specs=pl.BlockSpec((RB, D), lambda i: (i, 0)),
        out_shape=jax.ShapeDtypeStruct((n_pad, D), jnp.float32),
    )(x_pad, W1, deg_part)


def _tc_layer2(p0, p1, xs1, deg_part, W2, b1, n_pad):
    def body(p0_ref, p1_ref, xs_ref, deg_ref, w_ref, b_ref, o_ref):
        dinv = _dinv_of(deg_ref)
        h = jnp.maximum(
            dinv * (p0_ref[...] + p1_ref[...] + xs_ref[...]) + b_ref[...], 0.0)
        o_ref[...] = jnp.dot(
            h, w_ref[...], preferred_element_type=jnp.float32) * dinv

    return pl.pallas_call(
        body,
        grid=(n_pad // RB,),
        in_specs=[
            pl.BlockSpec((RB, D), lambda i: (i, 0)),
            pl.BlockSpec((RB, D), lambda i: (i, 0)),
            pl.BlockSpec((RB, D), lambda i: (i, 0)),
            pl.BlockSpec((NC, RB), lambda i: (0, i)),
            pl.BlockSpec((D, D), lambda i: (0, 0)),
            pl.BlockSpec((1, D), lambda i: (0, 0)),
        ],
        out_specs=pl.BlockSpec((RB, D), lambda i: (i, 0)),
        out_shape=jax.ShapeDtypeStruct((n_pad, D), jnp.float32),
    )(p0, p1, xs1, deg_part, W2, b1)


def _tc_head(p0, p1, xs2, deg_part, b2, P1, pb1, P2, pb2, y_pad, m_pad, n_pad):
    od = P2.shape[1]

    def body(p0_ref, p1_ref, xs_ref, deg_ref, b2_ref, P1_ref, pb1_ref,
             P2_ref, pb2_ref, y_ref, m_ref, lo_ref, s_ref, c_ref):
        dinv = _dinv_of(deg_ref)
        ne = dinv * (p0_ref[...] + p1_ref[...] + xs_ref[...]) + b2_ref[...]
        z = jnp.maximum(
            jnp.dot(ne, P1_ref[...], preferred_element_type=jnp.float32)
            + pb1_ref[...], 0.0)
        logits = jnp.dot(
            z, P2_ref[...], preferred_element_type=jnp.float32) + pb2_ref[...]
        lo_ref[...] = logits
        m = m_ref[...]
        bce = (jnp.maximum(logits, 0.0) - logits * y_ref[...]
               + jnp.log1p(jnp.exp(-jnp.abs(logits))))

        @pl.when(pl.program_id(0) == 0)
        def _():
            s_ref[...] = jnp.zeros_like(s_ref)
            c_ref[...] = jnp.zeros_like(c_ref)

        s_ref[...] += jnp.sum(bce * m)
        c_ref[...] += jnp.sum(m)

    return pl.pallas_call(
        body,
        grid=(n_pad // RB,),
        in_specs=[
            pl.BlockSpec((RB, D), lambda i: (i, 0)),
            pl.BlockSpec((RB, D), lambda i: (i, 0)),
            pl.BlockSpec((RB, D), lambda i: (i, 0)),
            pl.BlockSpec((NC, RB), lambda i: (0, i)),
            pl.BlockSpec((1, D), lambda i: (0, 0)),
            pl.BlockSpec((D, D), lambda i: (0, 0)),
            pl.BlockSpec((1, D), lambda i: (0, 0)),
            pl.BlockSpec((D, od), lambda i: (0, 0)),
            pl.BlockSpec((1, od), lambda i: (0, 0)),
            pl.BlockSpec((RB, od), lambda i: (i, 0)),
            pl.BlockSpec((RB, 1), lambda i: (i, 0)),
        ],
        out_specs=[
            pl.BlockSpec((RB, od), lambda i: (i, 0)),
            pl.BlockSpec((1, 1), lambda i: (0, 0)),
            pl.BlockSpec((1, 1), lambda i: (0, 0)),
        ],
        out_shape=[
            jax.ShapeDtypeStruct((n_pad, od), jnp.float32),
            jax.ShapeDtypeStruct((1, 1), jnp.float32),
            jax.ShapeDtypeStruct((1, 1), jnp.float32),
        ],
    )(p0, p1, xs2, deg_part, b2, P1, pb1, P2, pb2, y_pad, m_pad)


def kernel(x, edge_index, train_mask, target_labels,
           W1, b1, W2, b2, P1, pb1, P2, pb2):
    n = x.shape[0]
    n_pad = -(-n // RB) * RB
    if n_pad == n:
        n_pad += RB
    e = edge_index.shape[1]
    cpt = -(-e // (NW * CH))
    cpt = -(-cpt // NB) * NB
    e_pad = NW * CH * cpt
    pad = e_pad - e

    pad_src = np.arange(pad, dtype=np.int32) % n
    pad_dst = n + np.arange(pad, dtype=np.int32) % (n_pad - n)
    src_all = jnp.concatenate([edge_index[0], jnp.asarray(pad_src)])
    dst_all = jnp.concatenate([edge_index[1], jnp.asarray(pad_dst)])
    packed2 = (src_all | (dst_all << 16)).reshape(NW, cpt * CH)
    dst3 = dst_all.reshape(NW, cpt, CH)

    x_pad = jnp.pad(x, ((0, n_pad - n), (0, 0)))
    zeros = jnp.zeros((n_pad, D), jnp.float32)

    deg_part = _sc_degree(dst3, n_pad)
    xs1 = _tc_embed1(x_pad, W1, deg_part, n_pad)
    part1 = _sc_scatter(xs1, packed2, zeros, n_pad)
    xs2 = _tc_layer2(part1[0], part1[1], xs1, deg_part,
                     W2, b1.reshape(1, D), n_pad)
    part2 = _sc_scatter(xs2, packed2, zeros, n_pad)

    y_pad = jnp.pad(target_labels, ((0, n_pad - n), (0, 0)))
    m_pad = jnp.pad(train_mask.astype(jnp.float32),
                    (0, n_pad - n)).reshape(n_pad, 1)
    logits_pad, s, c = _tc_head(part2[0], part2[1], xs2, deg_part,
                                b2.reshape(1, D), P1, pb1.reshape(1, -1),
                                P2, pb2.reshape(1, -1), y_pad, m_pad, n_pad)
    logits = logits_pad[:n]
    loss = s[0, 0] / (c[0, 0] * logits.shape[1])
    return (logits, loss)


# speedup vs baseline: 1.0452x; 1.0452x over previous
import functools

import numpy as np

import jax
import jax.numpy as jnp
from jax import lax
from jax.experimental import pallas as pl
from jax.experimental.pallas import tpu as pltpu
from jax.experimental.pallas import tpu_sc as plsc

D = 128
HD = D // 2
LN = 16
NC = 2
NS = 16
CH = 128
RB = 1024


def _sc_degree(dst3, n_pad):
    cpt = dst3.shape[1]
    rpt = n_pad // NS
    mesh = plsc.VectorSubcoreMesh(core_axis_name="c", subcore_axis_name="s")

    @functools.partial(
        pl.kernel,
        out_type=jax.ShapeDtypeStruct((NC, n_pad), jnp.float32),
        mesh=mesh,
        scratch_types=[
            pltpu.VMEM((cpt, CH), jnp.int32),
            pltpu.VMEM((CH,), jnp.float32),
            pltpu.VMEM((rpt,), jnp.float32),
            pltpu.VMEM_SHARED((n_pad,), jnp.float32),
        ],
        compiler_params=pltpu.CompilerParams(use_tc_tiling_on_sc=False),
    )
    def deg_kernel(dst_hbm, out_hbm, dst_v, ones_v, zbuf, acc):
        cid = lax.axis_index("c")
        sid = lax.axis_index("s")
        pltpu.sync_copy(dst_hbm.at[sid], dst_v)

        def zb(i, c):
            zbuf[pl.ds(i * LN, LN)] = jnp.zeros((LN,), jnp.float32)
            return c

        lax.fori_loop(0, rpt // LN, zb, 0)

        def ob(i, c):
            ones_v[pl.ds(i * LN, LN)] = jnp.ones((LN,), jnp.float32)
            return c

        lax.fori_loop(0, CH // LN, ob, 0)
        pltpu.sync_copy(zbuf, acc.at[pl.ds(sid * rpt, rpt)])
        plsc.subcore_barrier()

        def body(i, k):
            c = 2 * i + cid
            pltpu.sync_copy(ones_v, acc.at[dst_v.at[c]], add=True)
            return k

        lax.fori_loop(0, cpt // 2, body, 0)
        plsc.subcore_barrier()
        pltpu.sync_copy(acc.at[pl.ds(sid * rpt, rpt)],
                        out_hbm.at[cid, pl.ds(sid * rpt, rpt)])

    return deg_kernel(dst3)


def _sc_scatter(xs2v, src2, dst3, zeros, n_pad):
    cpt = dst3.shape[1]
    ept = cpt * CH
    rpt = n_pad // NS
    mesh = plsc.VectorSubcoreMesh(core_axis_name="c", subcore_axis_name="s")
    nb = 5

    @functools.partial(
        pl.kernel,
        out_type=jax.ShapeDtypeStruct((n_pad, NC, HD), jnp.float32),
        mesh=mesh,
        scratch_types=[
            pltpu.VMEM((ept,), jnp.int32),
            pltpu.VMEM((cpt, CH), jnp.int32),
            [pltpu.VMEM((CH, HD), jnp.float32)] * nb,
            pltpu.VMEM_SHARED((n_pad, HD), jnp.float32),
            [pltpu.SemaphoreType.DMA] * nb,
            [pltpu.SemaphoreType.DMA] * nb,
        ],
        compiler_params=pltpu.CompilerParams(use_tc_tiling_on_sc=False),
    )
    def scat_kernel(xs_hbm, src_hbm, dst_hbm, zero_hbm, out_hbm,
                    src_v, dst_v, bufs, acc, gsem, ssem):
        cid = lax.axis_index("c")
        sid = lax.axis_index("s")
        r0 = sid * rpt
        pltpu.sync_copy(src_hbm.at[cid, sid], src_v)
        pltpu.sync_copy(dst_hbm.at[sid], dst_v)
        pltpu.sync_copy(zero_hbm.at[pl.ds(r0, rpt)], acc.at[pl.ds(r0, rpt)])
        plsc.subcore_barrier()

        def gather(c, b):
            return pltpu.make_async_copy(
                xs_hbm.at[src_v.at[pl.ds(c * CH, CH)]], bufs[b], gsem[b])

        def scat_start(c, b):
            pltpu.async_copy(bufs[b], acc.at[dst_v.at[c]], ssem[b], add=True)

        def scat_wait(c, b):
            pltpu.make_async_copy(bufs[b], acc.at[dst_v.at[c]], ssem[b]).wait()

        for b in range(nb):
            gather(b, b).start()

        def body(j, k):
            c = nb * j
            for b in range(nb):
                gather(c + b, b).wait()
                scat_start(c + b, b)
            for b in range(nb):
                scat_wait(c + b, b)
                gather(jnp.minimum(c + nb + b, cpt - 1), b).start()
            return k

        lax.fori_loop(0, cpt // nb, body, 0)
        for b in range(nb):
            gather(0, b).wait()
        plsc.subcore_barrier()
        pltpu.sync_copy(acc.at[pl.ds(r0, rpt)],
                        out_hbm.at[pl.ds(r0, rpt), cid])

    return scat_kernel(xs2v, src2, dst3, zeros)


def _dinv_of(deg_ref):
    deg = deg_ref[0, :] + deg_ref[1, :] + 1.0
    return lax.rsqrt(deg)[:, None]


def _tc_embed1(x_pad, W1, deg_part, n_pad):
    def body(x_ref, w_ref, deg_ref, o_ref):
        xw = jnp.dot(x_ref[...], w_ref[...], preferred_element_type=jnp.float32)
        o_ref[...] = xw * _dinv_of(deg_ref)

    return pl.pallas_call(
        body,
        grid=(n_pad // RB,),
        in_specs=[
            pl.BlockSpec((RB, D), lambda i: (i, 0)),
            pl.BlockSpec((D, D), lambda i: (0, 0)),
            pl.BlockSpec((NC, RB), lambda i: (0, i)),
        ],
        out_specs=pl.BlockSpec((RB, D), lambda i: (i, 0)),
        out_shape=jax.ShapeDtypeStruct((n_pad, D), jnp.float32),
    )(x_pad, W1, deg_part)


def _tc_layer2(p1, xs1, deg_part, W2, b1, n_pad):
    def body(p_ref, xs_ref, deg_ref, w_ref, b_ref, o_ref):
        dinv = _dinv_of(deg_ref)
        h = jnp.maximum(
            dinv * (p_ref[...] + xs_ref[...]) + b_ref[...], 0.0)
        o_ref[...] = jnp.dot(
            h, w_ref[...], preferred_element_type=jnp.float32) * dinv

    return pl.pallas_call(
        body,
        grid=(n_pad // RB,),
        in_specs=[
            pl.BlockSpec((RB, D), lambda i: (i, 0)),
            pl.BlockSpec((RB, D), lambda i: (i, 0)),
            pl.BlockSpec((NC, RB), lambda i: (0, i)),
            pl.BlockSpec((D, D), lambda i: (0, 0)),
            pl.BlockSpec((1, D), lambda i: (0, 0)),
        ],
        out_specs=pl.BlockSpec((RB, D), lambda i: (i, 0)),
        out_shape=jax.ShapeDtypeStruct((n_pad, D), jnp.float32),
    )(p1, xs1, deg_part, W2, b1)


def _tc_head(p2, xs2, deg_part, b2, P1, pb1, P2, pb2, y_pad, m_pad, n_pad):
    od = P2.shape[1]

    def body(p_ref, xs_ref, deg_ref, b2_ref, P1_ref, pb1_ref,
             P2_ref, pb2_ref, y_ref, m_ref, lo_ref, s_ref, c_ref):
        dinv = _dinv_of(deg_ref)
        ne = dinv * (p_ref[...] + xs_ref[...]) + b2_ref[...]
        z = jnp.maximum(
            jnp.dot(ne, P1_ref[...], preferred_element_type=jnp.float32)
            + pb1_ref[...], 0.0)
        logits = jnp.dot(
            z, P2_ref[...], preferred_element_type=jnp.float32) + pb2_ref[...]
        lo_ref[...] = logits
        m = m_ref[...]
        bce = (jnp.maximum(logits, 0.0) - logits * y_ref[...]
               + jnp.log1p(jnp.exp(-jnp.abs(logits))))

        @pl.when(pl.program_id(0) == 0)
        def _():
            s_ref[...] = jnp.zeros_like(s_ref)
            c_ref[...] = jnp.zeros_like(c_ref)

        s_ref[...] += jnp.sum(bce * m)
        c_ref[...] += jnp.sum(m)

    return pl.pallas_call(
        body,
        grid=(n_pad // RB,),
        in_specs=[
            pl.BlockSpec((RB, D), lambda i: (i, 0)),
            pl.BlockSpec((RB, D), lambda i: (i, 0)),
            pl.BlockSpec((NC, RB), lambda i: (0, i)),
            pl.BlockSpec((1, D), lambda i: (0, 0)),
            pl.BlockSpec((D, D), lambda i: (0, 0)),
            pl.BlockSpec((1, D), lambda i: (0, 0)),
            pl.BlockSpec((D, od), lambda i: (0, 0)),
            pl.BlockSpec((1, od), lambda i: (0, 0)),
            pl.BlockSpec((RB, od), lambda i: (i, 0)),
            pl.BlockSpec((RB, 1), lambda i: (i, 0)),
        ],
        out_specs=[
            pl.BlockSpec((RB, od), lambda i: (i, 0)),
            pl.BlockSpec((1, 1), lambda i: (0, 0)),
            pl.BlockSpec((1, 1), lambda i: (0, 0)),
        ],
        out_shape=[
            jax.ShapeDtypeStruct((n_pad, od), jnp.float32),
            jax.ShapeDtypeStruct((1, 1), jnp.float32),
            jax.ShapeDtypeStruct((1, 1), jnp.float32),
        ],
    )(p2, xs2, deg_part, b2, P1, pb1, P2, pb2, y_pad, m_pad)


def kernel(x, edge_index, train_mask, target_labels,
           W1, b1, W2, b2, P1, pb1, P2, pb2):
    n = x.shape[0]
    n_pad = -(-n // RB) * RB
    if n_pad == n:
        n_pad += RB
    e = edge_index.shape[1]
    cpt = -(-e // (NS * CH))
    cpt = -(-cpt // 10) * 10
    e_pad = NS * CH * cpt
    pad = e_pad - e

    pad_src = np.arange(pad, dtype=np.int32) % n
    pad_dst = n + np.arange(pad, dtype=np.int32) % (n_pad - n)
    src_all = jnp.concatenate([edge_index[0], jnp.asarray(pad_src)])
    dst_all = jnp.concatenate([edge_index[1], jnp.asarray(pad_dst)])
    s2 = (2 * src_all).reshape(1, NS, cpt * CH)
    src2 = jnp.concatenate([s2, s2 + 1], axis=0)
    dst3 = dst_all.reshape(NS, cpt, CH)

    x_pad = jnp.pad(x, ((0, n_pad - n), (0, 0)))
    zeros = jnp.zeros((n_pad, HD), jnp.float32)

    deg_part = _sc_degree(dst3, n_pad)
    xs1 = _tc_embed1(x_pad, W1, deg_part, n_pad)
    part1 = _sc_scatter(xs1.reshape(2 * n_pad, HD), src2, dst3, zeros, n_pad)
    xs2 = _tc_layer2(part1.reshape(n_pad, D), xs1, deg_part,
                     W2, b1.reshape(1, D), n_pad)
    part2 = _sc_scatter(xs2.reshape(2 * n_pad, HD), src2, dst3, zeros, n_pad)

    y_pad = jnp.pad(target_labels, ((0, n_pad - n), (0, 0)))
    m_pad = jnp.pad(train_mask.astype(jnp.float32),
                    (0, n_pad - n)).reshape(n_pad, 1)
    logits_pad, s, c = _tc_head(part2.reshape(n_pad, D), xs2, deg_part,
                                b2.reshape(1, D), P1, pb1.reshape(1, -1),
                                P2, pb2.reshape(1, -1), y_pad, m_pad, n_pad)
    logits = logits_pad[:n]
    loss = s[0, 0] / (c[0, 0] * logits.shape[1])
    return (logits, loss)


# speedup vs baseline: 1.2703x; 1.2153x over previous
import functools

import numpy as np

import jax
import jax.numpy as jnp
from jax import lax
from jax.experimental import pallas as pl
from jax.experimental.pallas import tpu as pltpu
from jax.experimental.pallas import tpu_sc as plsc

D = 128
HD = D // 2
LN = 16
NC = 2
NS = 16
CH = 128
RB = 1024


def _sc_degree(dst3, n_pad):
    cpt = dst3.shape[1]
    rpt = n_pad // NS
    mesh = plsc.VectorSubcoreMesh(core_axis_name="c", subcore_axis_name="s")

    @functools.partial(
        pl.kernel,
        out_type=jax.ShapeDtypeStruct((NC, n_pad), jnp.float32),
        mesh=mesh,
        scratch_types=[
            pltpu.VMEM((cpt, CH), jnp.int32),
            pltpu.VMEM((CH,), jnp.float32),
            pltpu.VMEM((rpt,), jnp.float32),
            pltpu.VMEM_SHARED((n_pad,), jnp.float32),
        ],
        compiler_params=pltpu.CompilerParams(use_tc_tiling_on_sc=False),
    )
    def deg_kernel(dst_hbm, out_hbm, dst_v, ones_v, zbuf, acc):
        cid = lax.axis_index("c")
        sid = lax.axis_index("s")
        pltpu.sync_copy(dst_hbm.at[sid], dst_v)

        def zb(i, c):
            zbuf[pl.ds(i * LN, LN)] = jnp.zeros((LN,), jnp.float32)
            return c

        lax.fori_loop(0, rpt // LN, zb, 0)

        def ob(i, c):
            ones_v[pl.ds(i * LN, LN)] = jnp.ones((LN,), jnp.float32)
            return c

        lax.fori_loop(0, CH // LN, ob, 0)
        pltpu.sync_copy(zbuf, acc.at[pl.ds(sid * rpt, rpt)])
        plsc.subcore_barrier()

        def body(i, k):
            c = 2 * i + cid
            pltpu.sync_copy(ones_v, acc.at[dst_v.at[c]], add=True)
            return k

        lax.fori_loop(0, cpt // 2, body, 0)
        plsc.subcore_barrier()
        pltpu.sync_copy(acc.at[pl.ds(sid * rpt, rpt)],
                        out_hbm.at[cid, pl.ds(sid * rpt, rpt)])

    return deg_kernel(dst3)


def _sc_scatter(xs, src3, dst3, zeros, n_pad):
    cpt = src3.shape[1]
    rpt = n_pad // NS
    mesh = plsc.VectorSubcoreMesh(core_axis_name="c", subcore_axis_name="s")

    nb = 5

    @functools.partial(
        pl.kernel,
        out_type=jax.ShapeDtypeStruct((NC, n_pad, HD), jnp.float32),
        mesh=mesh,
        scratch_types=[
            pltpu.VMEM((cpt, CH), jnp.int32),
            pltpu.VMEM((cpt, CH), jnp.int32),
            [pltpu.VMEM((CH, HD), jnp.float32)] * nb,
            pltpu.VMEM_SHARED((n_pad, HD), jnp.float32),
            [pltpu.SemaphoreType.DMA] * nb,
            [pltpu.SemaphoreType.DMA] * nb,
        ],
        compiler_params=pltpu.CompilerParams(use_tc_tiling_on_sc=False),
    )
    def scat_kernel(xs_hbm, src_hbm, dst_hbm, zero_hbm, out_hbm,
                    src_v, dst_v, bufs, acc, gsem, ssem):
        cid = lax.axis_index("c")
        sid = lax.axis_index("s")
        r0 = sid * rpt
        tab = xs_hbm.at[cid]
        pltpu.sync_copy(src_hbm.at[sid], src_v)
        pltpu.sync_copy(dst_hbm.at[sid], dst_v)
        pltpu.sync_copy(zero_hbm.at[pl.ds(r0, rpt)], acc.at[pl.ds(r0, rpt)])
        plsc.subcore_barrier()

        def gather(c, b):
            return pltpu.make_async_copy(tab.at[src_v.at[c]], bufs[b], gsem[b])

        def scat_start(c, b):
            pltpu.async_copy(bufs[b], acc.at[dst_v.at[c]], ssem[b], add=True)

        def scat_wait(c, b):
            pltpu.make_async_copy(bufs[b], acc.at[dst_v.at[c]], ssem[b]).wait()

        for b in range(nb):
            gather(b, b).start()

        def body(j, k):
            c = nb * j
            for b in range(nb):
                gather(c + b, b).wait()
                scat_start(c + b, b)
            for b in range(nb):
                scat_wait(c + b, b)
                gather(jnp.minimum(c + nb + b, cpt - 1), b).start()
            return k

        lax.fori_loop(0, cpt // nb, body, 0)
        for b in range(nb):
            gather(0, b).wait()
        plsc.subcore_barrier()
        pltpu.sync_copy(acc.at[pl.ds(r0, rpt)],
                        out_hbm.at[cid, pl.ds(r0, rpt)])

    return scat_kernel(xs, src3, dst3, zeros)


def _dinv_of(deg_ref):
    deg = deg_ref[0, :] + deg_ref[1, :] + 1.0
    return lax.rsqrt(deg)[:, None]


def _eo_of(pair0, pair1):
    ev = jnp.concatenate([pair0[:, :HD], pair1[:, :HD]], axis=-1)
    od_ = jnp.concatenate([pair0[:, HD:], pair1[:, HD:]], axis=-1)
    return ev, od_


def _pair_store(o_ref, ev, od_):
    o_ref[0, :, :] = jnp.concatenate([ev[:, :HD], od_[:, :HD]], axis=-1)
    o_ref[1, :, :] = jnp.concatenate([ev[:, HD:], od_[:, HD:]], axis=-1)


def _tc_embed1(x2, W1, degE, degO, n_pad):
    def body(x_ref, w_ref, dE_ref, dO_ref, o_ref):
        dE = _dinv_of(dE_ref)
        dO = _dinv_of(dO_ref)
        xwE = jnp.dot(x_ref[:, :D], w_ref[...],
                      preferred_element_type=jnp.float32) * dE
        xwO = jnp.dot(x_ref[:, D:], w_ref[...],
                      preferred_element_type=jnp.float32) * dO
        _pair_store(o_ref, xwE, xwO)

    return pl.pallas_call(
        body,
        grid=(n_pad // RB,),
        in_specs=[
            pl.BlockSpec((RB // 2, 2 * D), lambda i: (i, 0)),
            pl.BlockSpec((D, D), lambda i: (0, 0)),
            pl.BlockSpec((NC, RB // 2), lambda i: (0, i)),
            pl.BlockSpec((NC, RB // 2), lambda i: (0, i)),
        ],
        out_specs=pl.BlockSpec((NC, RB // 2, D), lambda i: (0, i, 0)),
        out_shape=jax.ShapeDtypeStruct((NC, n_pad // 2, D), jnp.float32),
    )(x2, W1, degE, degO)


def _tc_layer2(part1, xs1, degE, degO, W2, b1, n_pad):
    def body(p_ref, xs_ref, dE_ref, dO_ref, w_ref, b_ref, o_ref):
        dE = _dinv_of(dE_ref)
        dO = _dinv_of(dO_ref)
        pE, pO = _eo_of(p_ref[0], p_ref[1])
        xE, xO = _eo_of(xs_ref[0], xs_ref[1])
        hE = jnp.maximum(dE * (pE + xE) + b_ref[...], 0.0)
        hO = jnp.maximum(dO * (pO + xO) + b_ref[...], 0.0)
        oE = jnp.dot(hE, w_ref[...], preferred_element_type=jnp.float32) * dE
        oO = jnp.dot(hO, w_ref[...], preferred_element_type=jnp.float32) * dO
        _pair_store(o_ref, oE, oO)

    return pl.pallas_call(
        body,
        grid=(n_pad // RB,),
        in_specs=[
            pl.BlockSpec((NC, RB // 2, D), lambda i: (0, i, 0)),
            pl.BlockSpec((NC, RB // 2, D), lambda i: (0, i, 0)),
            pl.BlockSpec((NC, RB // 2), lambda i: (0, i)),
            pl.BlockSpec((NC, RB // 2), lambda i: (0, i)),
            pl.BlockSpec((D, D), lambda i: (0, 0)),
            pl.BlockSpec((1, D), lambda i: (0, 0)),
        ],
        out_specs=pl.BlockSpec((NC, RB // 2, D), lambda i: (0, i, 0)),
        out_shape=jax.ShapeDtypeStruct((NC, n_pad // 2, D), jnp.float32),
    )(part1, xs1, degE, degO, W2, b1)


def _tc_head(part2, xs2, degE, degO, b2, P1, pb1, P2, pb2, y2, m2, n_pad):
    od = P2.shape[1]

    def bce_of(logits, y, m):
        b = (jnp.maximum(logits, 0.0) - logits * y
             + jnp.log1p(jnp.exp(-jnp.abs(logits))))
        return jnp.sum(b * m)

    def body(p_ref, xs_ref, dE_ref, dO_ref, b2_ref, P1_ref, pb1_ref,
             P2_ref, pb2_ref, y_ref, m_ref, lo_ref, s_ref, c_ref):
        dE = _dinv_of(dE_ref)
        dO = _dinv_of(dO_ref)
        pE, pO = _eo_of(p_ref[0], p_ref[1])
        xE, xO = _eo_of(xs_ref[0], xs_ref[1])
        neE = dE * (pE + xE) + b2_ref[...]
        neO = dO * (pO + xO) + b2_ref[...]
        zE = jnp.maximum(
            jnp.dot(neE, P1_ref[...], preferred_element_type=jnp.float32)
            + pb1_ref[...], 0.0)
        zO = jnp.maximum(
            jnp.dot(neO, P1_ref[...], preferred_element_type=jnp.float32)
            + pb1_ref[...], 0.0)
        lE = jnp.dot(
            zE, P2_ref[...], preferred_element_type=jnp.float32) + pb2_ref[...]
        lO = jnp.dot(
            zO, P2_ref[...], preferred_element_type=jnp.float32) + pb2_ref[...]
        lo_ref[...] = jnp.concatenate([lE, lO], axis=-1)

        @pl.when(pl.program_id(0) == 0)
        def _():
            s_ref[...] = jnp.zeros_like(s_ref)
            c_ref[...] = jnp.zeros_like(c_ref)

        s_ref[...] += (bce_of(lE, y_ref[:, :od], m_ref[:, 0:1])
                       + bce_of(lO, y_ref[:, od:], m_ref[:, 1:2]))
        c_ref[...] += jnp.sum(m_ref[...])

    return pl.pallas_call(
        body,
        grid=(n_pad // RB,),
        in_specs=[
            pl.BlockSpec((NC, RB // 2, D), lambda i: (0, i, 0)),
            pl.BlockSpec((NC, RB // 2, D), lambda i: (0, i, 0)),
            pl.BlockSpec((NC, RB // 2), lambda i: (0, i)),
            pl.BlockSpec((NC, RB // 2), lambda i: (0, i)),
            pl.BlockSpec((1, D), lambda i: (0, 0)),
            pl.BlockSpec((D, D), lambda i: (0, 0)),
            pl.BlockSpec((1, D), lambda i: (0, 0)),
            pl.BlockSpec((D, od), lambda i: (0, 0)),
            pl.BlockSpec((1, od), lambda i: (0, 0)),
            pl.BlockSpec((RB // 2, 2 * od), lambda i: (i, 0)),
            pl.BlockSpec((RB // 2, 2), lambda i: (i, 0)),
        ],
        out_specs=[
            pl.BlockSpec((RB // 2, 2 * od), lambda i: (i, 0)),
            pl.BlockSpec((1, 1), lambda i: (0, 0)),
            pl.BlockSpec((1, 1), lambda i: (0, 0)),
        ],
        out_shape=[
            jax.ShapeDtypeStruct((n_pad // 2, 2 * od), jnp.float32),
            jax.ShapeDtypeStruct((1, 1), jnp.float32),
            jax.ShapeDtypeStruct((1, 1), jnp.float32),
        ],
    )(part2, xs2, degE, degO, b2, P1, pb1, P2, pb2, y2, m2)


def kernel(x, edge_index, train_mask, target_labels,
           W1, b1, W2, b2, P1, pb1, P2, pb2):
    n = x.shape[0]
    n_pad = -(-n // RB) * RB
    if n_pad == n:
        n_pad += RB
    e = edge_index.shape[1]
    cpt = -(-e // (NS * CH))
    cpt = -(-cpt // 10) * 10
    e_pad = NS * CH * cpt
    pad = e_pad - e

    pad_src = np.arange(pad, dtype=np.int32) % n
    pad_dst = n + np.arange(pad, dtype=np.int32) % (n_pad - n)
    src3 = jnp.concatenate(
        [edge_index[0], jnp.asarray(pad_src)]).reshape(NS, cpt, CH)
    dst3 = jnp.concatenate(
        [edge_index[1], jnp.asarray(pad_dst)]).reshape(NS, cpt, CH)

    x_pad = jnp.pad(x, ((0, n_pad - n), (0, 0)))
    zeros = jnp.zeros((n_pad, HD), jnp.float32)

    def to_sc(a):
        return a.reshape(NC, n_pad, HD)

    def to_tc(a):
        return a.reshape(NC, n_pad // 2, D)

    deg_part = _sc_degree(dst3, n_pad)
    degE = deg_part[:, 0::2]
    degO = deg_part[:, 1::2]
    x2 = x_pad.reshape(n_pad // 2, 2 * D)
    xs1 = _tc_embed1(x2, W1, degE, degO, n_pad)
    part1 = _sc_scatter(to_sc(xs1), src3, dst3, zeros, n_pad)
    xs2 = _tc_layer2(to_tc(part1), xs1, degE, degO,
                     W2, b1.reshape(1, D), n_pad)
    part2 = _sc_scatter(to_sc(xs2), src3, dst3, zeros, n_pad)

    od = target_labels.shape[1]
    y2 = jnp.pad(target_labels,
                 ((0, n_pad - n), (0, 0))).reshape(n_pad // 2, 2 * od)
    m2 = jnp.pad(train_mask.astype(jnp.float32),
                 (0, n_pad - n)).reshape(n_pad // 2, 2)
    lo2, s, c = _tc_head(to_tc(part2), xs2, degE, degO,
                         b2.reshape(1, D), P1, pb1.reshape(1, -1),
                         P2, pb2.reshape(1, -1), y2, m2, n_pad)
    logits = lo2.reshape(n_pad, od)[:n]
    loss = s[0, 0] / (c[0, 0] * od)
    return (logits, loss)
